# Initial kernel scaffold; baseline (speedup 1.0000x reference)
import jax
import jax.numpy as jnp
from jax.experimental import pallas as pl


def kernel(attr, state, edge_index, Ra, pe_w0, pe_b0, pe_w1, pe_b1, re_w0, re_b0, re_w1, re_b1, re_w2, re_b2, rp_w, rp_b, pp_w, pp_b, pr_w0, pr_b0, pr_w1, pr_b1, pr_w2, pr_b2):
    raise NotImplementedError("write your pallas kernel here")


# speedup vs baseline: 3.4907x; 3.4907x over previous
import functools

import jax
import jax.numpy as jnp
from jax import lax
from jax.experimental import pallas as pl
from jax.experimental.pallas import tpu as pltpu
from jax.experimental.pallas import tpu_sc as plsc

N = 10000
E = 320000
NF = 128
POS = 3
EC = E // 128
NC = 2
NS = 16
NW = NC * NS
ROWS_W = EC // NW
EXTRA = EC - NW * ROWS_W
NPT = 632
NPAD = NPT * NS

_f32 = jnp.float32


def _node_pre_body(ps_ref, pw0_ref, pb0_ref, pw1_ref, pb1_ref,
                   wrr_ref, wrs_ref, rb0_ref, u1_ref, ppb_ref,
                   wr_ref, ws_ref,
                   pr_out, psn_out, pe2_out, a0_out, b0_out):
    ps = ps_ref[...]
    h = jnp.maximum(jnp.dot(ps, pw0_ref[...], preferred_element_type=_f32)
                    + pb0_ref[...], 0.0)
    pe = jnp.maximum(jnp.dot(h, pw1_ref[...], preferred_element_type=_f32)
                     + pb1_ref[...], 0.0)
    pr_out[...] = jnp.dot(ps, wrr_ref[...], preferred_element_type=_f32) + rb0_ref[...]
    psn_out[...] = jnp.dot(ps, wrs_ref[...], preferred_element_type=_f32)
    pe2_out[...] = jnp.dot(pe, u1_ref[...], preferred_element_type=_f32) + ppb_ref[...]
    a0_out[...] = jnp.dot(pe, wr_ref[...], preferred_element_type=_f32)
    b0_out[...] = jnp.dot(pe, ws_ref[...], preferred_element_type=_f32)


def _node_pre(ps, pw0, pb0, pw1, pb1, wrr, wrs, rb0, u1, ppb, wr, ws):
    bn = 1000
    grid = (N // bn,)
    row = pl.BlockSpec((bn, NF), lambda i: (i, 0))
    w16 = pl.BlockSpec((16, NF), lambda i: (0, 0))
    w128 = pl.BlockSpec((NF, NF), lambda i: (0, 0))
    bias = pl.BlockSpec((1, NF), lambda i: (0, 0))
    return pl.pallas_call(
        _node_pre_body,
        grid=grid,
        in_specs=[pl.BlockSpec((bn, 16), lambda i: (i, 0)),
                  w16, bias, w128, bias,
                  w16, w16, bias, w128, bias, w128, w128],
        out_specs=[row, row, row, row, row],
        out_shape=[jax.ShapeDtypeStruct((N, NF), _f32)] * 5,
    )(ps, pw0, pb0, pw1, pb1, wrr, wrs, rb0, u1, ppb, wr, ws)


def _edge_mlp_body(g_ref, ra_ref, wv_ref, w1_ref, b1_ref, w2_ref, b2_ref,
                   wre_ref, rb_ref, c_out):
    re0 = jnp.maximum(g_ref[...] + ra_ref[...] * wv_ref[...], 0.0)
    h = jnp.maximum(jnp.dot(re0, w1_ref[...], preferred_element_type=_f32)
                    + b1_ref[...], 0.0)
    h = jnp.maximum(jnp.dot(h, w2_ref[...], preferred_element_type=_f32)
                    + b2_ref[...], 0.0)
    c_out[...] = jnp.dot(h, wre_ref[...], preferred_element_type=_f32) + rb_ref[...]


def _edge_mlp(g, ra_col, wvec, w1, b1, w2, b2, wre, rb):
    be = 3200
    grid = (E // be,)
    row = pl.BlockSpec((be, NF), lambda i: (i, 0))
    w128 = pl.BlockSpec((NF, NF), lambda i: (0, 0))
    bias = pl.BlockSpec((1, NF), lambda i: (0, 0))
    return pl.pallas_call(
        _edge_mlp_body,
        grid=grid,
        in_specs=[row, pl.BlockSpec((be, 1), lambda i: (i, 0)), bias,
                  w128, bias, w128, bias, w128, bias],
        out_specs=row,
        out_shape=jax.ShapeDtypeStruct((E, NF), _f32),
    )(g, ra_col, wvec, w1, b1, w2, b2, wre, rb)


def _update_ab_body(aggp_ref, pe2_ref, u2_ref, wr_ref, ws_ref, a_out, b_out):
    agg = aggp_ref[0] + aggp_ref[1]
    eff = jnp.maximum(pe2_ref[...]
                      + jnp.dot(agg, u2_ref[...], preferred_element_type=_f32), 0.0)
    a_out[...] = jnp.dot(eff, wr_ref[...], preferred_element_type=_f32)
    b_out[...] = jnp.dot(eff, ws_ref[...], preferred_element_type=_f32)


def _update_ab(aggp, pe2, u2, wr, ws):
    bn = 1000
    grid = (N // bn,)
    row = pl.BlockSpec((bn, NF), lambda i: (i, 0))
    w128 = pl.BlockSpec((NF, NF), lambda i: (0, 0))
    return pl.pallas_call(
        _update_ab_body,
        grid=grid,
        in_specs=[pl.BlockSpec((2, bn, NF), lambda i: (0, i, 0)), row, w128, w128, w128],
        out_specs=[row, row],
        out_shape=[jax.ShapeDtypeStruct((N, NF), _f32)] * 2,
    )(aggp, pe2, u2, wr, ws)


def _update_pred_body(aggp_ref, pe2_ref, u2_ref, w0_ref, b0_ref,
                      w1_ref, b1_ref, w2_ref, b2_ref, p_out):
    agg = aggp_ref[0] + aggp_ref[1]
    eff = jnp.maximum(pe2_ref[...]
                      + jnp.dot(agg, u2_ref[...], preferred_element_type=_f32), 0.0)
    h = jnp.maximum(jnp.dot(eff, w0_ref[...], preferred_element_type=_f32)
                    + b0_ref[...], 0.0)
    h = jnp.maximum(jnp.dot(h, w1_ref[...], preferred_element_type=_f32)
                    + b1_ref[...], 0.0)
    p_out[...] = jnp.dot(h, w2_ref[...], preferred_element_type=_f32) + b2_ref[...]


def _update_pred(aggp, pe2, u2, w0, b0, w1, b1, w2, b2):
    bn = 1000
    grid = (N // bn,)
    row = pl.BlockSpec((bn, NF), lambda i: (i, 0))
    w128 = pl.BlockSpec((NF, NF), lambda i: (0, 0))
    bias = pl.BlockSpec((1, NF), lambda i: (0, 0))
    return pl.pallas_call(
        _update_pred_body,
        grid=grid,
        in_specs=[pl.BlockSpec((2, bn, NF), lambda i: (0, i, 0)), row, w128,
                  w128, bias, w128, bias,
                  pl.BlockSpec((NF, POS), lambda i: (0, 0)),
                  pl.BlockSpec((1, POS), lambda i: (0, 0))],
        out_specs=pl.BlockSpec((bn, POS), lambda i: (i, 0)),
        out_shape=jax.ShapeDtypeStruct((N, POS), _f32),
    )(aggp, pe2, u2, w0, b0, w1, b1, w2, b2)


def _sc_mesh():
    return plsc.VectorSubcoreMesh(core_axis_name="c", subcore_axis_name="s",
                                  num_cores=NC, num_subcores=NS)


def _worker_range(c, s):
    w = c * NS + s
    base = w * ROWS_W + jnp.minimum(w, EXTRA)
    cnt = ROWS_W + jnp.where(w < EXTRA, 1, 0)
    return base, cnt


def _sc_stage1(recv2, send2, pr, psn):
    @functools.partial(
        pl.kernel,
        out_type=jax.ShapeDtypeStruct((E, NF), _f32),
        mesh=_sc_mesh(),
        scratch_types=[
            pltpu.VMEM((128,), jnp.int32),
            pltpu.VMEM((128,), jnp.int32),
            pltpu.VMEM((128, NF), _f32),
            pltpu.VMEM((128, NF), _f32),
            pltpu.SemaphoreType.DMA,
            pltpu.SemaphoreType.DMA,
        ],
    )
    def k(recv_hbm, send_hbm, pr_hbm, ps_hbm, out_hbm,
          ridx, sidx, prv, psv, sem1, sem2):
        c = lax.axis_index("c")
        s = lax.axis_index("s")
        base, cnt = _worker_range(c, s)

        @pl.loop(base, base + cnt)
        def _row(row):
            pltpu.sync_copy(recv_hbm.at[row], ridx)
            pltpu.sync_copy(send_hbm.at[row], sidx)
            cp1 = pltpu.async_copy(pr_hbm.at[ridx], prv, sem1)
            cp2 = pltpu.async_copy(ps_hbm.at[sidx], psv, sem2)
            cp1.wait()
            cp2.wait()

            @pl.loop(0, 128)
            def _e(i):
                for j in range(8):
                    sl = pl.ds(j * 16, 16)
                    prv[i, sl] = prv[i, sl] + psv[i, sl]

            pltpu.sync_copy(prv, out_hbm.at[pl.ds(row * 128, 128)])

    return k(recv2, send2, pr, psn)


def _sc_step(c_arr, a_arr, b_arr, recv2, send2, zeros_n):
    @functools.partial(
        pl.kernel,
        out_type=jax.ShapeDtypeStruct((NC, NPAD, NF), _f32),
        mesh=_sc_mesh(),
        scratch_types=[
            pltpu.VMEM((128,), jnp.int32),
            pltpu.VMEM((128,), jnp.int32),
            pltpu.VMEM((128, NF), _f32),
            pltpu.VMEM((128, NF), _f32),
            pltpu.VMEM((128, NF), _f32),
            pltpu.VMEM_SHARED((NPAD, NF), _f32),
            pltpu.SemaphoreType.DMA,
            pltpu.SemaphoreType.DMA,
        ],
    )
    def k(c_hbm, a_hbm, b_hbm, recv_hbm, send_hbm, z_hbm, out_hbm,
          ridx, sidx, av, bv, cv, agg_sh, sem1, sem2):
        c = lax.axis_index("c")
        s = lax.axis_index("s")
        base, cnt = _worker_range(c, s)
        pltpu.sync_copy(z_hbm.at[pl.ds(s * NPT, NPT)], agg_sh.at[pl.ds(s * NPT, NPT)])
        plsc.subcore_barrier()

        @pl.loop(base, base + cnt)
        def _row(row):
            pltpu.sync_copy(recv_hbm.at[row], ridx)
            pltpu.sync_copy(send_hbm.at[row], sidx)
            cp1 = pltpu.async_copy(a_hbm.at[ridx], av, sem1)
            cp2 = pltpu.async_copy(b_hbm.at[sidx], bv, sem2)
            pltpu.sync_copy(c_hbm.at[pl.ds(row * 128, 128)], cv)
            cp1.wait()
            cp2.wait()

            @pl.loop(0, 128)
            def _e(i):
                for j in range(8):
                    sl = pl.ds(j * 16, 16)
                    cv[i, sl] = jnp.maximum(av[i, sl] + bv[i, sl] + cv[i, sl], 0.0)

            pltpu.sync_copy(cv, agg_sh.at[ridx], add=True)

        plsc.subcore_barrier()
        pltpu.sync_copy(agg_sh.at[pl.ds(s * NPT, NPT)],
                        out_hbm.at[c].at[pl.ds(s * NPT, NPT)])

    return k(c_arr, a_arr, b_arr, recv2, send2, zeros_n)


def kernel(attr, state, edge_index, Ra, pe_w0, pe_b0, pe_w1, pe_b1,
           re_w0, re_b0, re_w1, re_b1, re_w2, re_b2,
           rp_w, rp_b, pp_w, pp_b,
           pr_w0, pr_b0, pr_w1, pr_b1, pr_w2, pr_b2):
    zcol = jnp.zeros((N, 1), _f32)
    ps = jnp.concatenate([attr, state, zcol], axis=1)
    recv2 = edge_index[0].reshape(EC, 128)
    send2 = edge_index[1].reshape(EC, 128)

    zrow = jnp.zeros((1, NF), _f32)
    pw0 = jnp.concatenate([pe_w0, zrow], axis=0)
    wrr = jnp.concatenate([re_w0[0:15], zrow], axis=0)
    wrs = jnp.concatenate([re_w0[15:30], zrow], axis=0)
    wvec = re_w0[30]
    u1 = pp_w[0:NF]
    u2 = pp_w[NF:2 * NF]
    wre = rp_w[0:NF]
    wr = rp_w[NF:2 * NF]
    ws = rp_w[2 * NF:3 * NF]

    b = lambda x: x.reshape(1, -1)

    pr, psn, pe2, a, bb = _node_pre(ps, pw0, b(pe_b0), pe_w1, b(pe_b1),
                                    wrr, wrs, b(re_b0), u1, b(pp_b), wr, ws)
    g = _sc_stage1(recv2, send2, pr, psn)
    c_arr = _edge_mlp(g, Ra, b(wvec), re_w1, b(re_b1), re_w2, b(re_b2),
                      wre, b(rp_b))

    zeros_n = jnp.zeros((NPAD, NF), _f32)
    aggp = _sc_step(c_arr, a, bb, recv2, send2, zeros_n)
    a, bb = _update_ab(aggp, pe2, u2, wr, ws)
    aggp = _sc_step(c_arr, a, bb, recv2, send2, zeros_n)
    return _update_pred(aggp, pe2, u2, pr_w0, b(pr_b0),
                        pr_w1, b(pr_b1), pr_w2, b(pr_b2))


# speedup vs baseline: 4.7878x; 1.3716x over previous
import functools

import jax
import jax.numpy as jnp
import numpy as np
from jax import lax
from jax.experimental import pallas as pl
from jax.experimental.pallas import tpu as pltpu
from jax.experimental.pallas import tpu_sc as plsc

N = 10000
E = 320000
NF = 128
POS = 3
NC = 2
NS = 16
NW = NC * NS
NPT = 632
NPAD = NPT * NS
CB = 80
NCH = E // (NW * CB)
LUB = ((NCH + 2 + 3) // 4) * 4
CBS = 40
NCHS = E // (NW * CBS)
LUBS = ((NCHS + 2 + 3) // 4) * 4

_f32 = jnp.float32
_bf16 = jnp.bfloat16

_PERM = np.concatenate(
    [np.concatenate([32 * k + np.arange(0, 32, 2), 32 * k + np.arange(1, 32, 2)])
     for k in range(NF // 32)]
)


def _node_pre_body(ps_ref, pw0_ref, pb0_ref, pw1_ref, pb1_ref,
                   wrr_ref, wrs_ref, rb0_ref, u1_ref, ppb_ref,
                   wr_ref, ws_ref,
                   pr_out, psn_out, pe2_out, a0_out, b0_out):
    ps = ps_ref[...]
    h = jnp.maximum(jnp.dot(ps, pw0_ref[...], preferred_element_type=_f32)
                    + pb0_ref[...], 0.0)
    pe = jnp.maximum(jnp.dot(h, pw1_ref[...], preferred_element_type=_f32)
                     + pb1_ref[...], 0.0)
    pr_out[...] = jnp.dot(ps, wrr_ref[...], preferred_element_type=_f32) + rb0_ref[...]
    psn_out[...] = jnp.dot(ps, wrs_ref[...], preferred_element_type=_f32)
    pe2_out[...] = jnp.dot(pe, u1_ref[...], preferred_element_type=_f32) + ppb_ref[...]
    a0_out[...] = jnp.dot(pe, wr_ref[...], preferred_element_type=_f32)
    b0_out[...] = jnp.dot(pe, ws_ref[...], preferred_element_type=_f32)


def _node_pre(ps, pw0, pb0, pw1, pb1, wrr, wrs, rb0, u1, ppb, wr, ws):
    bn = 1000
    grid = (N // bn,)
    row = pl.BlockSpec((bn, NF), lambda i: (i, 0))
    w16 = pl.BlockSpec((16, NF), lambda i: (0, 0))
    w128 = pl.BlockSpec((NF, NF), lambda i: (0, 0))
    bias = pl.BlockSpec((1, NF), lambda i: (0, 0))
    return pl.pallas_call(
        _node_pre_body,
        grid=grid,
        in_specs=[pl.BlockSpec((bn, 16), lambda i: (i, 0)),
                  w16, bias, w128, bias,
                  w16, w16, bias, w128, bias, w128, w128],
        out_specs=[row, row, row, row, row],
        out_shape=[jax.ShapeDtypeStruct((N, NF), _f32)] * 5,
    )(ps, pw0, pb0, pw1, pb1, wrr, wrs, rb0, u1, ppb, wr, ws)


def _edge_mlp_body(g_ref, ra_ref, wv_ref, w1_ref, b1_ref, w2_ref, b2_ref,
                   wre_ref, rb_ref, c_out):
    re0 = jnp.maximum(g_ref[...] + ra_ref[...] * wv_ref[...], 0.0).astype(_bf16)
    h = jnp.maximum(jnp.dot(re0, w1_ref[...], preferred_element_type=_f32)
                    + b1_ref[...], 0.0).astype(_bf16)
    h = jnp.maximum(jnp.dot(h, w2_ref[...], preferred_element_type=_f32)
                    + b2_ref[...], 0.0).astype(_bf16)
    c_out[...] = jnp.dot(h, wre_ref[...], preferred_element_type=_f32) + rb_ref[...]


def _edge_mlp(g, ra_col, wvec, w1, b1, w2, b2, wre, rb):
    be = 3200
    grid = (E // be,)
    row = pl.BlockSpec((be, NF), lambda i: (i, 0))
    w128 = pl.BlockSpec((NF, NF), lambda i: (0, 0))
    bias = pl.BlockSpec((1, NF), lambda i: (0, 0))
    return pl.pallas_call(
        _edge_mlp_body,
        grid=grid,
        in_specs=[row, pl.BlockSpec((be, 1), lambda i: (i, 0)), bias,
                  w128, bias, w128, bias, w128, bias],
        out_specs=row,
        out_shape=jax.ShapeDtypeStruct((E, NF), _f32),
    )(g, ra_col, wvec, w1, b1, w2, b2, wre, rb)


def _update_ab_body(aggp_ref, pe2_ref, u2p_ref, wr_ref, ws_ref, a_out, b_out):
    agg = (aggp_ref[0] + aggp_ref[1]).astype(_bf16)
    eff = jnp.maximum(pe2_ref[...]
                      + jnp.dot(agg, u2p_ref[...], preferred_element_type=_f32),
                      0.0).astype(_bf16)
    a_out[...] = jnp.dot(eff, wr_ref[...], preferred_element_type=_f32)
    b_out[...] = jnp.dot(eff, ws_ref[...], preferred_element_type=_f32)


def _update_ab(aggp, pe2, u2p, wr, ws):
    bn = 1000
    grid = (N // bn,)
    row = pl.BlockSpec((bn, NF), lambda i: (i, 0))
    w128 = pl.BlockSpec((NF, NF), lambda i: (0, 0))
    return pl.pallas_call(
        _update_ab_body,
        grid=grid,
        in_specs=[pl.BlockSpec((2, bn, NF), lambda i: (0, i, 0)), row, w128, w128, w128],
        out_specs=[row, row],
        out_shape=[jax.ShapeDtypeStruct((N, NF), _f32)] * 2,
    )(aggp, pe2, u2p, wr, ws)


def _update_pred_body(aggp_ref, pe2_ref, u2p_ref, w0_ref, b0_ref,
                      w1_ref, b1_ref, w2_ref, b2_ref, p_out):
    agg = (aggp_ref[0] + aggp_ref[1]).astype(_bf16)
    eff = jnp.maximum(pe2_ref[...]
                      + jnp.dot(agg, u2p_ref[...], preferred_element_type=_f32),
                      0.0).astype(_bf16)
    h = jnp.maximum(jnp.dot(eff, w0_ref[...], preferred_element_type=_f32)
                    + b0_ref[...], 0.0).astype(_bf16)
    h = jnp.maximum(jnp.dot(h, w1_ref[...], preferred_element_type=_f32)
                    + b1_ref[...], 0.0).astype(_bf16)
    p_out[...] = jnp.dot(h, w2_ref[...], preferred_element_type=_f32) + b2_ref[...]


def _update_pred(aggp, pe2, u2p, w0, b0, w1, b1, w2, b2):
    bn = 1000
    grid = (N // bn,)
    row = pl.BlockSpec((bn, NF), lambda i: (i, 0))
    w128 = pl.BlockSpec((NF, NF), lambda i: (0, 0))
    bias = pl.BlockSpec((1, NF), lambda i: (0, 0))
    return pl.pallas_call(
        _update_pred_body,
        grid=grid,
        in_specs=[pl.BlockSpec((2, bn, NF), lambda i: (0, i, 0)), row, w128,
                  w128, bias, w128, bias,
                  pl.BlockSpec((NF, POS), lambda i: (0, 0)),
                  pl.BlockSpec((1, POS), lambda i: (0, 0))],
        out_specs=pl.BlockSpec((bn, POS), lambda i: (i, 0)),
        out_shape=jax.ShapeDtypeStruct((N, POS), _f32),
    )(aggp, pe2, u2p, w0, b0, w1, b1, w2, b2)


def _sc_mesh():
    return plsc.VectorSubcoreMesh(core_axis_name="c", subcore_axis_name="s",
                                  num_cores=NC, num_subcores=NS)


def _sc_stage1(recv1, send1, pr, psn):
    @functools.partial(
        pl.kernel,
        out_type=jax.ShapeDtypeStruct((E, NF), _f32),
        mesh=_sc_mesh(),
        scratch_types=(
            [pltpu.VMEM((CB,), jnp.int32)] * 8
            + [pltpu.VMEM((CB, NF), _f32)] * 6
            + [pltpu.SemaphoreType.DMA] * 8
        ),
    )
    def k(recv_hbm, send_hbm, pr_hbm, ps_hbm, out_hbm,
          ri0, ri1, ri2, ri3, si0, si1, si2, si3,
          pv0, pv1, sv0, sv1, ov0, ov1,
          smi0, smi1, smi2, smi3, smg0, smg1, smo0, smo1):
        ridx = [ri0, ri1, ri2, ri3]
        sidx = [si0, si1, si2, si3]
        pv = [pv0, pv1]
        sv = [sv0, sv1]
        ov = [ov0, ov1]
        sem_i = [smi0, smi1, smi2, smi3]
        sem_g = [smg0, smg1]
        sem_o = [smo0, smo1]
        c = lax.axis_index("c")
        s = lax.axis_index("s")
        w = c * NS + s
        e0 = w * (NCH * CB)

        def issue_idx(kk, sl):
            eb = e0 + kk * CB
            pltpu.async_copy(recv_hbm.at[pl.ds(eb, CB)], ridx[sl], sem_i[sl])
            pltpu.async_copy(send_hbm.at[pl.ds(eb, CB)], sidx[sl], sem_i[sl])

        def wait_idx(sl):
            pltpu.make_async_copy(recv_hbm.at[pl.ds(e0, CB)], ridx[sl], sem_i[sl]).wait()
            pltpu.make_async_copy(send_hbm.at[pl.ds(e0, CB)], sidx[sl], sem_i[sl]).wait()

        def issue_gather(kk, sl, b):
            pltpu.async_copy(pr_hbm.at[ridx[sl]], pv[b], sem_g[b])
            pltpu.async_copy(ps_hbm.at[sidx[sl]], sv[b], sem_g[b])

        def wait_gather(sl, b):
            pltpu.make_async_copy(pr_hbm.at[ridx[sl]], pv[b], sem_g[b]).wait()
            pltpu.make_async_copy(ps_hbm.at[sidx[sl]], sv[b], sem_g[b]).wait()

        issue_idx(0, 0)
        wait_idx(0)
        issue_gather(0, 0, 0)
        issue_idx(1, 1)

        @pl.loop(0, LUB, step=4)
        def _outer(k4):
            for b4 in range(4):
                kk = k4 + b4
                b = b4 % 2

                @pl.when(kk + 1 < NCH)
                def _():
                    wait_idx((b4 + 1) % 4)
                    issue_gather(kk + 1, (b4 + 1) % 4, 1 - b)

                @pl.when(kk < NCH)
                def _():
                    wait_gather(b4, b)

                @pl.when(kk + 2 < NCH)
                def _():
                    issue_idx(kk + 2, (b4 + 2) % 4)

                @pl.when((kk >= 2) & (kk < NCH + 2))
                def _():
                    pltpu.make_async_copy(
                        ov[b], out_hbm.at[pl.ds(e0, CB)], sem_o[b]).wait()

                @pl.when(kk < NCH)
                def _():
                    @pl.loop(0, CB)
                    def _e(i):
                        for j in range(8):
                            sl = pl.ds(j * 16, 16)
                            ov[b][i, sl] = pv[b][i, sl] + sv[b][i, sl]

                    pltpu.async_copy(ov[b], out_hbm.at[pl.ds(e0 + kk * CB, CB)],
                                     sem_o[b])

    return k(recv1, send1, pr, psn)


def _sc_step(c_arr, a_arr, b_arr, recv1, send1, zeros_n):
    @functools.partial(
        pl.kernel,
        out_type=jax.ShapeDtypeStruct((NC, NPAD, NF), _f32),
        mesh=_sc_mesh(),
        scratch_types=(
            [pltpu.VMEM((CBS,), jnp.int32)] * 8
            + [pltpu.VMEM((CBS, NF), _f32)] * 7
            + [pltpu.VMEM_SHARED((NPAD, NF), _f32)]
            + [pltpu.SemaphoreType.DMA] * 6
        ),
    )
    def k(c_hbm, a_hbm, b_hbm, recv_hbm, send_hbm, z_hbm, out_hbm,
          ri0, ri1, ri2, ri3, si0, si1, si2, si3,
          av0, av1, bv0, bv1, cv0, cv1, ev, agg_sh,
          smi0, smi1, smi2, smi3, smg0, smg1):
        ridx = [ri0, ri1, ri2, ri3]
        sidx = [si0, si1, si2, si3]
        av = [av0, av1]
        bv = [bv0, bv1]
        cv = [cv0, cv1]
        sem_i = [smi0, smi1, smi2, smi3]
        sem_g = [smg0, smg1]
        c = lax.axis_index("c")
        s = lax.axis_index("s")
        w = c * NS + s
        e0 = w * (NCHS * CBS)
        pltpu.sync_copy(z_hbm.at[pl.ds(s * NPT, NPT)], agg_sh.at[pl.ds(s * NPT, NPT)])
        plsc.subcore_barrier()

        def issue_idx(kk, sl):
            eb = e0 + kk * CBS
            pltpu.async_copy(recv_hbm.at[pl.ds(eb, CBS)], ridx[sl], sem_i[sl])
            pltpu.async_copy(send_hbm.at[pl.ds(eb, CBS)], sidx[sl], sem_i[sl])

        def wait_idx(sl):
            pltpu.make_async_copy(recv_hbm.at[pl.ds(e0, CBS)], ridx[sl], sem_i[sl]).wait()
            pltpu.make_async_copy(send_hbm.at[pl.ds(e0, CBS)], sidx[sl], sem_i[sl]).wait()

        def issue_gather(kk, sl, b):
            pltpu.async_copy(a_hbm.at[ridx[sl]], av[b], sem_g[b])
            pltpu.async_copy(b_hbm.at[sidx[sl]], bv[b], sem_g[b])
            pltpu.async_copy(c_hbm.at[pl.ds(e0 + kk * CBS, CBS)], cv[b], sem_g[b])

        def wait_gather(sl, b):
            pltpu.make_async_copy(a_hbm.at[ridx[sl]], av[b], sem_g[b]).wait()
            pltpu.make_async_copy(b_hbm.at[sidx[sl]], bv[b], sem_g[b]).wait()
            pltpu.make_async_copy(c_hbm.at[pl.ds(e0, CBS)], cv[b], sem_g[b]).wait()

        issue_idx(0, 0)
        wait_idx(0)
        issue_gather(0, 0, 0)
        issue_idx(1, 1)

        @pl.loop(0, LUBS, step=4)
        def _outer(k4):
            for b4 in range(4):
                kk = k4 + b4
                b = b4 % 2

                @pl.when(kk + 1 < NCHS)
                def _():
                    wait_idx((b4 + 1) % 4)
                    issue_gather(kk + 1, (b4 + 1) % 4, 1 - b)

                @pl.when(kk < NCHS)
                def _():
                    wait_gather(b4, b)

                @pl.when(kk < NCHS)
                def _():
                    @pl.loop(0, CBS)
                    def _e(i):
                        for j in range(8):
                            sl = pl.ds(j * 16, 16)
                            ev[i, sl] = jnp.maximum(
                                av[b][i, sl] + bv[b][i, sl] + cv[b][i, sl], 0.0)

                    pltpu.sync_copy(ev, agg_sh.at[ridx[b4]], add=True)

                @pl.when(kk + 2 < NCHS)
                def _():
                    issue_idx(kk + 2, (b4 + 2) % 4)

        plsc.subcore_barrier()
        pltpu.sync_copy(agg_sh.at[pl.ds(s * NPT, NPT)],
                        out_hbm.at[c].at[pl.ds(s * NPT, NPT)])

    return k(c_arr, a_arr, b_arr, recv1, send1, zeros_n)


def kernel(attr, state, edge_index, Ra, pe_w0, pe_b0, pe_w1, pe_b1,
           re_w0, re_b0, re_w1, re_b1, re_w2, re_b2,
           rp_w, rp_b, pp_w, pp_b,
           pr_w0, pr_b0, pr_w1, pr_b1, pr_w2, pr_b2):
    zcol = jnp.zeros((N, 1), _f32)
    ps = jnp.concatenate([attr, state, zcol], axis=1)
    recv1 = edge_index[0]
    send1 = edge_index[1]

    zrow = jnp.zeros((1, NF), _f32)
    pw0 = jnp.concatenate([pe_w0, zrow], axis=0)
    wrr = jnp.concatenate([re_w0[0:15], zrow], axis=0)
    wrs = jnp.concatenate([re_w0[15:30], zrow], axis=0)
    wvec = re_w0[30]
    u1 = pp_w[0:NF]
    u2p = pp_w[NF:2 * NF].astype(_bf16)
    wre = rp_w[0:NF].astype(_bf16)
    wr = rp_w[NF:2 * NF]
    ws = rp_w[2 * NF:3 * NF]

    b = lambda x: x.reshape(1, -1)

    pr, psn, pe2, a, bb = _node_pre(ps, pw0, b(pe_b0), pe_w1, b(pe_b1),
                                    wrr, wrs, b(re_b0), u1, b(pp_b), wr, ws)
    g = _sc_stage1(recv1, send1, pr, psn)
    c_arr = _edge_mlp(g, Ra, b(wvec), re_w1.astype(_bf16), b(re_b1),
                      re_w2.astype(_bf16), b(re_b2), wre, b(rp_b))

    zeros_n = jnp.zeros((NPAD, NF), _f32)
    aggp = _sc_step(c_arr, a, bb, recv1, send1, zeros_n)
    a, bb = _update_ab(aggp, pe2, u2p, wr.astype(_bf16), ws.astype(_bf16))
    aggp = _sc_step(c_arr, a, bb, recv1, send1, zeros_n)
    return _update_pred(aggp, pe2, u2p, pr_w0.astype(_bf16), b(pr_b0),
                        pr_w1.astype(_bf16), b(pr_b1),
                        pr_w2.astype(_bf16), b(pr_b2))
